# baseline (speedup 1.0000x reference)
import functools

import jax
import jax.numpy as jnp
from jax import lax
from jax.experimental import pallas as pl
from jax.experimental.pallas import tpu as pltpu
from jax.experimental.pallas import tpu_sc as plsc

N = 50000
E = 800000
D_NODE = 16
D_REV = 64

NC = 2
NS = 16
NW = NC * NS

NPAD = 50048

CHUNK = 1136
PER_W = 24992
NCHUNK = PER_W // CHUNK
TAIL = E - NW * PER_W
TAIL_E0 = NW * PER_W
ROWS_PER_TILE = NPAD // NS

BE = 6400


def _h_body(x_ref, w_ref, o_ref):
    o_ref[...] = jnp.dot(x_ref[...], w_ref[...], preferred_element_type=jnp.float32)


def _rfw_body(rf_ref, w_ref, ew_ref, o_ref):
    o_ref[...] = (
        jnp.dot(rf_ref[...], w_ref[...], preferred_element_type=jnp.float32)
        * ew_ref[...]
    )


def _add_body(a_ref, b_ref, o_ref):
    o_ref[...] = a_ref[...] + b_ref[...]


_sc_mesh = plsc.VectorSubcoreMesh(core_axis_name="c", subcore_axis_name="s")


@functools.partial(
    pl.kernel,
    out_type=jax.ShapeDtypeStruct((NC, NPAD, D_NODE), jnp.float32),
    mesh=_sc_mesh,
    scratch_types=[
        pltpu.VMEM((1, CHUNK), jnp.int32),
        pltpu.VMEM((1, CHUNK), jnp.int32),
        pltpu.VMEM((CHUNK, 1), jnp.float32),
        pltpu.VMEM((CHUNK, D_NODE), jnp.float32),
        pltpu.VMEM((CHUNK, D_NODE), jnp.float32),
        pltpu.VMEM((1, TAIL), jnp.int32),
        pltpu.VMEM((1, TAIL), jnp.int32),
        pltpu.VMEM((TAIL, 1), jnp.float32),
        pltpu.VMEM((TAIL, D_NODE), jnp.float32),
        pltpu.VMEM((TAIL, D_NODE), jnp.float32),
        pltpu.VMEM_SHARED((NPAD, D_NODE), jnp.float32),
        pltpu.SemaphoreType.DMA,
    ],
    compiler_params=pltpu.CompilerParams(use_tc_tiling_on_sc=False, needs_layout_passes=False),
)
def _sc_scatter(h_hbm, rfw_hbm, ei_hbm, w_hbm, zeros_hbm, out_hbm,
                sidx_v, didx_v, w_v, g_v, rfw_v,
                sidx_t, didx_t, w_t, g_t, rfw_t, acc, sem):

    def process(e0, n, sidx, didx, wv, gv, rfwv):
        pltpu.sync_copy(ei_hbm.at[pl.ds(0, 1), pl.ds(e0, n)], sidx)
        pltpu.sync_copy(ei_hbm.at[pl.ds(1, 1), pl.ds(e0, n)], didx)
        pltpu.sync_copy(w_hbm.at[pl.ds(e0, n)], wv)
        pltpu.sync_copy(rfw_hbm.at[pl.ds(e0, n)], rfwv)
        pltpu.async_copy(h_hbm.at[sidx.at[0]], gv, sem).wait()

        lane = lax.iota(jnp.int32, 16)
        zero16 = jnp.zeros((16,), jnp.int32)

        def body(k, _):
            i0 = k * 16
            wvec = plsc.load_gather(wv, [i0 + lane, zero16])
            for j in range(16):
                i = i0 + j
                gv[i, :] = gv[i, :] * wvec[j] + rfwv[i, :]
            return 0

        lax.fori_loop(0, n // 16, body, 0)
        pltpu.sync_copy(gv, acc.at[didx.at[0]], add=True)

    cid = lax.axis_index("c")
    sid = lax.axis_index("s")
    wid = cid * NS + sid

    pltpu.sync_copy(zeros_hbm, acc.at[pl.ds(sid * ROWS_PER_TILE, ROWS_PER_TILE)])
    plsc.subcore_barrier()

    base_e = wid * PER_W

    for g in range(NCHUNK):
        process(base_e + g * CHUNK, CHUNK, sidx_v, didx_v, w_v, g_v, rfw_v)

    @pl.when(wid == NW - 1)
    def _():
        process(TAIL_E0, TAIL, sidx_t, didx_t, w_t, g_t, rfw_t)

    plsc.subcore_barrier()
    pltpu.sync_copy(acc.at[pl.ds(sid * ROWS_PER_TILE, ROWS_PER_TILE)],
                    out_hbm.at[cid, pl.ds(sid * ROWS_PER_TILE, ROWS_PER_TILE)])


def kernel(x, edge_index, review_feat, edge_w, node_W, review_W):
    zeros = jnp.zeros((ROWS_PER_TILE, D_NODE), jnp.float32)

    h = pl.pallas_call(
        _h_body,
        out_shape=jax.ShapeDtypeStruct((N, D_NODE), jnp.float32),
    )(x, node_W)

    rfw = pl.pallas_call(
        _rfw_body,
        grid=(E // BE,),
        in_specs=[
            pl.BlockSpec((BE, D_REV), lambda i: (i, 0)),
            pl.BlockSpec((D_REV, D_NODE), lambda i: (0, 0)),
            pl.BlockSpec((BE, 1), lambda i: (i, 0)),
        ],
        out_specs=pl.BlockSpec((BE, D_NODE), lambda i: (i, 0)),
        out_shape=jax.ShapeDtypeStruct((E, D_NODE), jnp.float32),
    )(review_feat, review_W, edge_w)

    partials = _sc_scatter(h, rfw, edge_index, edge_w, zeros)

    rst = pl.pallas_call(
        _add_body,
        grid=(10,),
        in_specs=[
            pl.BlockSpec((N // 10, D_NODE), lambda i: (i, 0)),
            pl.BlockSpec((N // 10, D_NODE), lambda i: (i, 0)),
        ],
        out_specs=pl.BlockSpec((N // 10, D_NODE), lambda i: (i, 0)),
        out_shape=jax.ShapeDtypeStruct((N, D_NODE), jnp.float32),
    )(partials[0], partials[1])
    return rst


# speedup vs baseline: 2.2127x; 2.2127x over previous
import functools

import jax
import jax.numpy as jnp
from jax import lax
from jax.experimental import pallas as pl
from jax.experimental.pallas import tpu as pltpu
from jax.experimental.pallas import tpu_sc as plsc

N = 50000
E = 800000
D_NODE = 16
D_REV = 64

NC = 2
NS = 16
NW = NC * NS

NPAD = 50048

CHUNK = 1136
PER_W = 24992
NCHUNK = PER_W // CHUNK
TAIL = E - NW * PER_W
TAIL_E0 = NW * PER_W
ROWS_PER_TILE = NPAD // NS

BR = 1000


def _h_body(x_ref, w_ref, o_ref):
    o_ref[...] = jnp.dot(x_ref[...], w_ref[...], preferred_element_type=jnp.float32)


def _rf_body(rf2_ref, wblk_ref, o_ref):
    o_ref[...] = jnp.dot(rf2_ref[...], wblk_ref[...],
                         preferred_element_type=jnp.float32)


def _add_body(a_ref, b_ref, o_ref):
    o_ref[...] = a_ref[...] + b_ref[...]


_sc_mesh = plsc.VectorSubcoreMesh(core_axis_name="c", subcore_axis_name="s")


@functools.partial(
    pl.kernel,
    out_type=jax.ShapeDtypeStruct((NC, NPAD, D_NODE), jnp.float32),
    mesh=_sc_mesh,
    scratch_types=[
        pltpu.VMEM((CHUNK,), jnp.int32),
        pltpu.VMEM((CHUNK,), jnp.int32),
        pltpu.VMEM((CHUNK,), jnp.float32),
        pltpu.VMEM((CHUNK, D_NODE), jnp.float32),
        pltpu.VMEM((CHUNK // 8, 128), jnp.float32),
        pltpu.VMEM((TAIL,), jnp.int32),
        pltpu.VMEM((TAIL,), jnp.int32),
        pltpu.VMEM((TAIL,), jnp.float32),
        pltpu.VMEM((TAIL, D_NODE), jnp.float32),
        pltpu.VMEM((TAIL // 8, 128), jnp.float32),
        pltpu.VMEM_SHARED((NPAD, D_NODE), jnp.float32),
        pltpu.SemaphoreType.DMA,
    ],
    compiler_params=pltpu.CompilerParams(use_tc_tiling_on_sc=False,
                                         needs_layout_passes=False),
)
def _sc_scatter(h_hbm, rf_hbm, src_hbm, dst_hbm, w_hbm, zeros_hbm, out_hbm,
                sidx_v, didx_v, w_v, g_v, rf_v,
                sidx_t, didx_t, w_t, g_t, rf_t, acc, sem):

    def process(e0, n, sidx, didx, wv, gv, rfv):
        pltpu.sync_copy(src_hbm.at[pl.ds(e0, n)], sidx)
        pltpu.sync_copy(dst_hbm.at[pl.ds(e0, n)], didx)
        pltpu.sync_copy(w_hbm.at[pl.ds(e0, n)], wv)
        pltpu.sync_copy(rf_hbm.at[pl.ds(e0 // 8, n // 8)], rfv)
        pltpu.async_copy(h_hbm.at[sidx], gv, sem).wait()

        def body(k, _):
            i0 = k * 16
            wvec = wv[pl.ds(i0, 16)]
            r0 = 2 * k
            for j in range(16):
                i = i0 + j
                r = rfv[r0 + j // 8, pl.ds(16 * (j % 8), 16)]
                gv[i, :] = (gv[i, :] + r) * wvec[j]
            return 0

        lax.fori_loop(0, n // 16, body, 0)
        pltpu.sync_copy(gv, acc.at[didx], add=True)

    cid = lax.axis_index("c")
    sid = lax.axis_index("s")
    wid = cid * NS + sid

    pltpu.sync_copy(zeros_hbm, acc.at[pl.ds(sid * ROWS_PER_TILE, ROWS_PER_TILE)])
    plsc.subcore_barrier()

    base_e = wid * PER_W

    for g in range(NCHUNK):
        process(base_e + g * CHUNK, CHUNK, sidx_v, didx_v, w_v, g_v, rf_v)

    @pl.when(wid == NW - 1)
    def _():
        process(TAIL_E0, TAIL, sidx_t, didx_t, w_t, g_t, rf_t)

    plsc.subcore_barrier()
    pltpu.sync_copy(acc.at[pl.ds(sid * ROWS_PER_TILE, ROWS_PER_TILE)],
                    out_hbm.at[cid, pl.ds(sid * ROWS_PER_TILE, ROWS_PER_TILE)])


def kernel(x, edge_index, review_feat, edge_w, node_W, review_W):
    zeros = jnp.zeros((ROWS_PER_TILE, D_NODE), jnp.float32)
    src = edge_index[0]
    dst = edge_index[1]
    w_flat = edge_w.reshape(E)
    rf2 = review_feat.reshape(E // 8, 8 * D_REV)
    wblk = jnp.kron(jnp.eye(8, dtype=jnp.float32), review_W)

    h = pl.pallas_call(
        _h_body,
        out_shape=jax.ShapeDtypeStruct((N, D_NODE), jnp.float32),
    )(x, node_W)

    rf = pl.pallas_call(
        _rf_body,
        grid=(E // 8 // BR,),
        in_specs=[
            pl.BlockSpec((BR, 8 * D_REV), lambda i: (i, 0)),
            pl.BlockSpec((8 * D_REV, 8 * D_NODE), lambda i: (0, 0)),
        ],
        out_specs=pl.BlockSpec((BR, 8 * D_NODE), lambda i: (i, 0)),
        out_shape=jax.ShapeDtypeStruct((E // 8, 8 * D_NODE), jnp.float32),
    )(rf2, wblk)

    partials = _sc_scatter(h, rf, src, dst, w_flat, zeros)

    rst = pl.pallas_call(
        _add_body,
        grid=(10,),
        in_specs=[
            pl.BlockSpec((N // 10, D_NODE), lambda i: (i, 0)),
            pl.BlockSpec((N // 10, D_NODE), lambda i: (i, 0)),
        ],
        out_specs=pl.BlockSpec((N // 10, D_NODE), lambda i: (i, 0)),
        out_shape=jax.ShapeDtypeStruct((N, D_NODE), jnp.float32),
    )(partials[0], partials[1])
    return rst
